# baseline (speedup 1.0000x reference)
import dataclasses
import functools

import jax
import jax.numpy as jnp
from jax import lax
from jax.experimental import pallas as pl
from jax.experimental.pallas import tpu as pltpu
from jax.experimental.pallas import tpu_sc as plsc

N = 10000
E = 320000
NC, NS = 2, 16
NW = NC * NS
EPW = E // NW
CHUNK = 128
NCHUNK = 80
EPAD = NCHUNK * CHUNK - EPW
NDUMMY = 8
NHCH = EPW // 16
RPS = N // NS
NBUF = 8

F32 = jnp.float32


def _mesh():
    return plsc.VectorSubcoreMesh(
        core_axis_name="c", subcore_axis_name="s",
        num_cores=NC, num_subcores=NS)


def _sc_params():
    cp = pltpu.CompilerParams()
    fields = pltpu.CompilerParams.__dataclass_fields__
    if "needs_layout_passes" in fields:
        cp = dataclasses.replace(cp, needs_layout_passes=False)
    if "use_tc_tiling_on_sc" in fields:
        cp = dataclasses.replace(cp, use_tc_tiling_on_sc=False)
    return cp


BR = 2000
NB = N // BR


def _hist(dst16):
    @functools.partial(
        pl.kernel,
        out_type=jax.ShapeDtypeStruct((NB, NW, BR), F32),
        mesh=_mesh(),
        scratch_types=[
            pltpu.VMEM((NB, BR), F32),
            pltpu.VMEM((NHCH, 16), jnp.int32),
        ],
        compiler_params=_sc_params(),
    )
    def k(dst_hbm, out_hbm, hist, idx):
        c = lax.axis_index("c")
        s = lax.axis_index("s")
        w = s * NC + c
        pltpu.sync_copy(dst_hbm.at[w], idx)

        for i in range(NB):
            @pl.loop(0, BR, step=16)
            def _(j):
                hist.at[i, pl.ds(j, 16)][...] = jnp.zeros((16,), F32)

        ones = jnp.ones((16,), F32)

        @pl.loop(0, NHCH)
        def _(j):
            d = idx.at[j][...]
            row = d // BR
            col = d - row * BR
            plsc.addupdate_scatter(hist, [row, col], ones)

        for i in range(NB):
            pltpu.sync_copy(hist.at[i], out_hbm.at[i, w])

    return k(dst16)


def _agg(g, srcc, dstc, zeros, C, dt):
    @functools.partial(
        pl.kernel,
        out_type=jax.ShapeDtypeStruct((NC, NS, RPS, C), dt),
        mesh=_mesh(),
        scratch_types=[
            pltpu.VMEM((NCHUNK, CHUNK), jnp.int32),
            pltpu.VMEM((NCHUNK, CHUNK), jnp.int32),
        ] + [pltpu.VMEM((CHUNK, C), dt)] * NBUF + [
            pltpu.VMEM_SHARED((N + NDUMMY, C), dt),
        ] + [pltpu.SemaphoreType.DMA] * (2 * NBUF),
        compiler_params=_sc_params(),
    )
    def k(g_hbm, src_hbm, dst_hbm, z_hbm, out_hbm, srcv, dstv, *rest):
        bufs = rest[:NBUF]
        acc = rest[NBUF]
        gsems = rest[NBUF + 1:2 * NBUF + 1]
        ssems = rest[2 * NBUF + 1:]
        c = lax.axis_index("c")
        s = lax.axis_index("s")
        w = s * NC + c

        def start_g(j, b):
            pltpu.async_copy(g_hbm.at[srcv.at[j]], bufs[b], gsems[b])

        def wait_g(b):
            pltpu.make_async_copy(g_hbm.at[srcv.at[0]], bufs[b],
                                  gsems[b]).wait()

        def start_s(j, b):
            pltpu.async_copy(bufs[b], acc.at[dstv.at[j]], ssems[b], add=True)

        def wait_s(b):
            pltpu.make_async_copy(bufs[b], acc.at[dstv.at[0]],
                                  ssems[b]).wait()

        pltpu.sync_copy(src_hbm.at[w], srcv)
        pltpu.sync_copy(dst_hbm.at[w], dstv)
        for b in range(NBUF):
            start_g(b, b)
        r0 = s * RPS
        pltpu.sync_copy(z_hbm.at[s], acc.at[pl.ds(r0, RPS)])
        plsc.subcore_barrier()

        @pl.loop(0, NCHUNK - 2 * NBUF, step=NBUF)
        def _(k4):
            for b in range(NBUF):
                j = k4 + b
                wait_g(b)
                start_s(j, b)
                wait_s(b)
                start_g(j + NBUF, b)

        for b in range(NBUF):
            j = NCHUNK - 2 * NBUF + b
            wait_g(b)
            start_s(j, b)
            wait_s(b)
            start_g(j + NBUF, b)
        for b in range(NBUF):
            wait_g(b)
            start_s(NCHUNK - NBUF + b, b)
            wait_s(b)

        plsc.subcore_barrier()
        pltpu.sync_copy(acc.at[pl.ds(r0, RPS)], out_hbm.at[c, s])

    return k(g, srcc, dstc, zeros)


def _prep(x, W, counts3):
    def body(x_ref, w_ref, c_ref, dinv_ref, gb_ref):
        deg = 1.0 + jnp.sum(c_ref[0], axis=0)
        dinv = lax.rsqrt(deg)[:, None]
        dinv_ref[...] = dinv
        g = jnp.dot(x_ref[...], w_ref[...],
                    preferred_element_type=F32) * dinv
        gb_ref[...] = g.astype(jnp.bfloat16)

    C = W.shape[1]
    return pl.pallas_call(
        body,
        grid=(NB,),
        in_specs=[pl.BlockSpec((BR, x.shape[1]), lambda i: (i, 0)),
                  pl.BlockSpec(W.shape, lambda i: (0, 0)),
                  pl.BlockSpec((1, NW, BR), lambda i: (i, 0, 0))],
        out_specs=(pl.BlockSpec((BR, 1), lambda i: (i, 0)),
                   pl.BlockSpec((BR, C), lambda i: (i, 0))),
        out_shape=(jax.ShapeDtypeStruct((N, 1), F32),
                   jax.ShapeDtypeStruct((N, C), jnp.bfloat16)),
    )(x, W, counts3)


def _layer(acc, g, dinv, b2d, W, out_dt):
    def body(a_ref, g_ref, d_ref, b_ref, w_ref, o_ref):
        a = (a_ref[0, 0] + a_ref[0, 1]).astype(F32)
        t = (a + g_ref[...].astype(F32)) * d_ref[...] + b_ref[...]
        z = jnp.maximum(t, 0.0)
        o = jnp.dot(z, w_ref[...], preferred_element_type=F32) * d_ref[...]
        o_ref[...] = o.astype(out_dt)

    Cin = g.shape[1]
    C = W.shape[1]
    acc4 = acc.reshape(1, NC, N, Cin)
    return pl.pallas_call(
        body,
        grid=(NB,),
        in_specs=[pl.BlockSpec((1, NC, BR, Cin), lambda i: (0, 0, i, 0)),
                  pl.BlockSpec((BR, Cin), lambda i: (i, 0)),
                  pl.BlockSpec((BR, 1), lambda i: (i, 0)),
                  pl.BlockSpec((1, Cin), lambda i: (0, 0)),
                  pl.BlockSpec((Cin, C), lambda i: (0, 0))],
        out_specs=pl.BlockSpec((BR, C), lambda i: (i, 0)),
        out_shape=jax.ShapeDtypeStruct((N, C), out_dt),
    )(acc4, g, dinv, b2d, W)


def _final(acc, g, dinv, b2d):
    def body(a_ref, g_ref, d_ref, b_ref, o_ref):
        a = (a_ref[0, 0] + a_ref[0, 1]).astype(F32)
        o_ref[...] = (a + g_ref[...]) * d_ref[...] + b_ref[...]

    C = g.shape[1]
    acc4 = acc.reshape(1, NC, N, C)
    return pl.pallas_call(
        body,
        grid=(NB,),
        in_specs=[pl.BlockSpec((1, NC, BR, C), lambda i: (0, 0, i, 0)),
                  pl.BlockSpec((BR, C), lambda i: (i, 0)),
                  pl.BlockSpec((BR, 1), lambda i: (i, 0)),
                  pl.BlockSpec((1, C), lambda i: (0, 0))],
        out_specs=pl.BlockSpec((BR, C), lambda i: (i, 0)),
        out_shape=jax.ShapeDtypeStruct(g.shape, F32),
    )(acc4, g, dinv, b2d)


def kernel(x, edge_index, W1, b1, W2, b2, W3, b3):
    src = edge_index[0].astype(jnp.int32)
    dst = edge_index[1].astype(jnp.int32)
    pad_src = jnp.broadcast_to((jnp.arange(EPAD, dtype=jnp.int32) * 41) % N,
                               (NW, EPAD))
    pad_dst = jnp.broadcast_to(N + (jnp.arange(EPAD, dtype=jnp.int32)
                                    % NDUMMY), (NW, EPAD))
    srcc = jnp.concatenate([src.reshape(NW, EPW), pad_src],
                           axis=1).reshape(NW, NCHUNK, CHUNK)
    dstc = jnp.concatenate([dst.reshape(NW, EPW), pad_dst],
                           axis=1).reshape(NW, NCHUNK, CHUNK)
    dst16 = dst.reshape(NW, NHCH, 16)
    BF16 = jnp.bfloat16
    z64 = jnp.zeros((NS, RPS, 64), BF16)
    z16 = jnp.zeros((NS, RPS, 16), F32)

    counts3 = _hist(dst16)
    dinv, g1 = _prep(x, W1, counts3)
    acc1 = _agg(g1, srcc, dstc, z64, 64, BF16).reshape(NC, N, 64)
    g2 = _layer(acc1, g1, dinv, b1.reshape(1, -1), W2, BF16)
    acc2 = _agg(g2, srcc, dstc, z64, 64, BF16).reshape(NC, N, 64)
    g3 = _layer(acc2, g2, dinv, b2.reshape(1, -1), W3, F32)
    acc3 = _agg(g3, srcc, dstc, z16, 16, F32).reshape(NC, N, 16)
    return _final(acc3, g3, dinv, b3.reshape(1, -1))


# speedup vs baseline: 1.1424x; 1.1424x over previous
import dataclasses
import functools

import jax
import jax.numpy as jnp
from jax import lax
from jax.experimental import pallas as pl
from jax.experimental.pallas import tpu as pltpu
from jax.experimental.pallas import tpu_sc as plsc

N = 10000
E = 320000
NC, NS = 2, 16
NW = NC * NS
EPW = E // NW
CHUNK = 128
NCHUNK = 80
EPAD = NCHUNK * CHUNK - EPW
NDUMMY = 8
NHCH = EPW // 16
RPS = N // NS
NBUF = 8

F32 = jnp.float32


def _mesh():
    return plsc.VectorSubcoreMesh(
        core_axis_name="c", subcore_axis_name="s",
        num_cores=NC, num_subcores=NS)


def _sc_params():
    cp = pltpu.CompilerParams()
    fields = pltpu.CompilerParams.__dataclass_fields__
    if "needs_layout_passes" in fields:
        cp = dataclasses.replace(cp, needs_layout_passes=False)
    if "use_tc_tiling_on_sc" in fields:
        cp = dataclasses.replace(cp, use_tc_tiling_on_sc=False)
    return cp


def _hist(dst16):
    @functools.partial(
        pl.kernel,
        out_type=jax.ShapeDtypeStruct((NW, 1, N), F32),
        mesh=_mesh(),
        scratch_types=[
            pltpu.VMEM((N,), F32),
            pltpu.VMEM((NHCH, 16), jnp.int32),
        ],
        compiler_params=_sc_params(),
    )
    def k(dst_hbm, out_hbm, hist, idx):
        c = lax.axis_index("c")
        s = lax.axis_index("s")
        w = s * NC + c
        pltpu.sync_copy(dst_hbm.at[w], idx)

        @pl.loop(0, N, step=16)
        def _(i):
            hist.at[pl.ds(i, 16)][...] = jnp.zeros((16,), F32)

        ones = jnp.ones((16,), F32)

        @pl.loop(0, NHCH)
        def _(j):
            plsc.addupdate_scatter(hist, [idx.at[j][...]], ones)

        pltpu.sync_copy(hist, out_hbm.at[w, 0])

    return k(dst16)


def _agg(g, srcc, dstc, zeros, C, dt):
    @functools.partial(
        pl.kernel,
        out_type=jax.ShapeDtypeStruct((NC, NS, RPS, C), dt),
        mesh=_mesh(),
        scratch_types=[
            pltpu.VMEM((NCHUNK, CHUNK), jnp.int32),
            pltpu.VMEM((NCHUNK, CHUNK), jnp.int32),
        ] + [pltpu.VMEM((CHUNK, C), dt)] * NBUF + [
            pltpu.VMEM_SHARED((N + NDUMMY, C), dt),
        ] + [pltpu.SemaphoreType.DMA] * (2 * NBUF),
        compiler_params=_sc_params(),
    )
    def k(g_hbm, src_hbm, dst_hbm, z_hbm, out_hbm, srcv, dstv, *rest):
        bufs = rest[:NBUF]
        acc = rest[NBUF]
        gsems = rest[NBUF + 1:2 * NBUF + 1]
        ssems = rest[2 * NBUF + 1:]
        c = lax.axis_index("c")
        s = lax.axis_index("s")
        w = s * NC + c

        def start_g(j, b):
            pltpu.async_copy(g_hbm.at[srcv.at[j]], bufs[b], gsems[b])

        def wait_g(b):
            pltpu.make_async_copy(g_hbm.at[srcv.at[0]], bufs[b],
                                  gsems[b]).wait()

        def start_s(j, b):
            pltpu.async_copy(bufs[b], acc.at[dstv.at[j]], ssems[b], add=True)

        def wait_s(b):
            pltpu.make_async_copy(bufs[b], acc.at[dstv.at[0]],
                                  ssems[b]).wait()

        pltpu.sync_copy(src_hbm.at[w], srcv)
        pltpu.sync_copy(dst_hbm.at[w], dstv)
        for b in range(NBUF):
            start_g(b, b)
        r0 = s * RPS
        pltpu.sync_copy(z_hbm.at[s], acc.at[pl.ds(r0, RPS)])
        plsc.subcore_barrier()

        @pl.loop(0, NCHUNK - 2 * NBUF, step=NBUF)
        def _(k4):
            for b in range(NBUF):
                j = k4 + b
                wait_g(b)
                start_s(j, b)
                wait_s(b)
                start_g(j + NBUF, b)

        for b in range(NBUF):
            j = NCHUNK - 2 * NBUF + b
            wait_g(b)
            start_s(j, b)
            wait_s(b)
            start_g(j + NBUF, b)
        for b in range(NBUF):
            wait_g(b)
            start_s(NCHUNK - NBUF + b, b)
            wait_s(b)

        plsc.subcore_barrier()
        pltpu.sync_copy(acc.at[pl.ds(r0, RPS)], out_hbm.at[c, s])

    return k(g, srcc, dstc, zeros)


def _prep(x, W, counts):
    def body(x_ref, w_ref, c_ref, dinv_ref, gb_ref):
        deg = 1.0 + jnp.sum(c_ref[...], axis=0)
        dinv = lax.rsqrt(deg)[:, None]
        dinv_ref[...] = dinv
        g = jnp.dot(x_ref[...], w_ref[...],
                    preferred_element_type=F32) * dinv
        gb_ref[...] = g.astype(jnp.bfloat16)

    C = W.shape[1]
    return pl.pallas_call(
        body,
        out_shape=(jax.ShapeDtypeStruct((N, 1), F32),
                   jax.ShapeDtypeStruct((N, C), jnp.bfloat16)),
    )(x, W, counts)


def _layer(acc, g, dinv, b2d, W, out_dt):
    def body(a_ref, g_ref, d_ref, b_ref, w_ref, o_ref):
        a = (a_ref[0] + a_ref[1]).astype(F32)
        t = (a + g_ref[...].astype(F32)) * d_ref[...] + b_ref[...]
        z = jnp.maximum(t, 0.0)
        o = jnp.dot(z, w_ref[...], preferred_element_type=F32) * d_ref[...]
        o_ref[...] = o.astype(out_dt)

    C = W.shape[1]
    return pl.pallas_call(
        body,
        out_shape=jax.ShapeDtypeStruct((N, C), out_dt),
    )(acc, g, dinv, b2d, W)


def _final(acc, g, dinv, b2d):
    def body(a_ref, g_ref, d_ref, b_ref, o_ref):
        a = (a_ref[0] + a_ref[1]).astype(F32)
        o_ref[...] = (a + g_ref[...]) * d_ref[...] + b_ref[...]

    return pl.pallas_call(
        body,
        out_shape=jax.ShapeDtypeStruct(g.shape, F32),
    )(acc, g, dinv, b2d)


def kernel(x, edge_index, W1, b1, W2, b2, W3, b3):
    src = edge_index[0].astype(jnp.int32)
    dst = edge_index[1].astype(jnp.int32)
    pad_src = jnp.broadcast_to((jnp.arange(EPAD, dtype=jnp.int32) * 41) % N,
                               (NW, EPAD))
    pad_dst = jnp.broadcast_to(N + (jnp.arange(EPAD, dtype=jnp.int32)
                                    % NDUMMY), (NW, EPAD))
    srcc = jnp.concatenate([src.reshape(NW, EPW), pad_src],
                           axis=1).reshape(NW, NCHUNK, CHUNK)
    dstc = jnp.concatenate([dst.reshape(NW, EPW), pad_dst],
                           axis=1).reshape(NW, NCHUNK, CHUNK)
    dst16 = dst.reshape(NW, NHCH, 16)
    BF16 = jnp.bfloat16
    z64 = jnp.zeros((NS, RPS, 64), BF16)
    z16 = jnp.zeros((NS, RPS, 16), F32)

    counts = _hist(dst16).reshape(NW, N)
    dinv, g1 = _prep(x, W1, counts)
    acc1 = _agg(g1, srcc, dstc, z64, 64, BF16).reshape(NC, N, 64)
    g2 = _layer(acc1, g1, dinv, b1.reshape(1, -1), W2, BF16)
    acc2 = _agg(g2, srcc, dstc, z64, 64, BF16).reshape(NC, N, 64)
    g3 = _layer(acc2, g2, dinv, b2.reshape(1, -1), W3, F32)
    acc3 = _agg(g3, srcc, dstc, z16, 16, F32).reshape(NC, N, 16)
    return _final(acc3, g3, dinv, b3.reshape(1, -1))
